# baseline (speedup 1.0000x reference)
import functools
import jax
import jax.numpy as jnp
from jax.experimental import pallas as pl
from jax.experimental.pallas import tpu as pltpu

N_ENT = 4096
N_REL = 200
DIM = 64
B = 128
NIDX = 4 * B
ALPHA = 0.1
MARGIN = 1.0
CHUNK = 64
NCHUNK = NIDX // CHUNK


def _onehot_rows(idx_col, n_cols, dtype=jnp.float32):
    r = idx_col.shape[0]
    cols = jax.lax.broadcasted_iota(jnp.int32, (r, n_cols), 1)
    return jnp.where(cols == idx_col, jnp.float32(1.0), jnp.float32(0.0)).astype(dtype)


def _rescal_losses(ent, relf, h_i, t_i, nh_i, nt_i, r_i):
    h = jnp.dot(_onehot_rows(h_i, N_ENT), ent, preferred_element_type=jnp.float32)
    t = jnp.dot(_onehot_rows(t_i, N_ENT), ent, preferred_element_type=jnp.float32)
    nh = jnp.dot(_onehot_rows(nh_i, N_ENT), ent, preferred_element_type=jnp.float32)
    nt = jnp.dot(_onehot_rows(nt_i, N_ENT), ent, preferred_element_type=jnp.float32)
    rg = jnp.dot(_onehot_rows(r_i, N_REL), relf, preferred_element_type=jnp.float32)

    rows64 = jax.lax.broadcasted_iota(jnp.int32, (DIM, DIM * DIM), 0)
    colmod = jax.lax.broadcasted_iota(jnp.int32, (DIM, DIM * DIM), 1) % DIM
    tile_m = jnp.where(colmod == rows64, jnp.float32(1.0), jnp.float32(0.0))
    segrows = jax.lax.broadcasted_iota(jnp.int32, (DIM * DIM, DIM), 0) // DIM
    segcols = jax.lax.broadcasted_iota(jnp.int32, (DIM * DIM, DIM), 1)
    seg_m = jnp.where(segrows == segcols, jnp.float32(1.0), jnp.float32(0.0))

    def score(hv, tv):
        trep = jnp.dot(tv, tile_m, preferred_element_type=jnp.float32)
        tmp = jnp.dot(rg * trep, seg_m, preferred_element_type=jnp.float32)
        return jnp.sum(hv * tmp, axis=1)

    pos = score(h, t)
    neg = score(nh, nt)
    return jnp.mean(jax.nn.relu(MARGIN + neg - pos))


def _body(idx1_sp, p_row, ent0, ent1, rel0f, rel1f, idx1c, idx2c,
          h0, t0, nh0, nt0, r0, h1, t1, nh1, nt1, r1,
          out, w_s, u_s, rows_s, misc_s, acc_s):
    i = pl.program_id(0)

    @pl.when(i == 0)
    def _setup():
        acc_s[0, 0] = jnp.float32(0.0)
        o1 = _onehot_rows(idx1c[...], N_ENT)
        e1 = jnp.dot(o1, ent0[...], preferred_element_type=jnp.float32)
        o2 = _onehot_rows(idx2c[...], N_ENT)
        e2 = jnp.dot(o2, ent1[...], preferred_element_type=jnp.float32)
        a = jnp.sum(e1 * e1, axis=1, keepdims=True)
        b = jnp.sum(e2 * e2, axis=1, keepdims=True)
        ones = jnp.ones((NIDX, 1), jnp.float32)
        zeros = jnp.zeros((NIDX, 128 - 2 - DIM), jnp.float32)
        c_mat = jnp.concatenate([ones, b, e2, zeros], axis=1)
        u_mat = jnp.concatenate([a, ones, -2.0 * e1, zeros], axis=1)
        u_s[...] = u_mat
        rows_iota = jax.lax.broadcasted_iota(jnp.int32, (N_ENT, NIDX), 0)
        o2t = jnp.where(rows_iota == jnp.reshape(idx2c[...], (1, NIDX)),
                        jnp.float32(1.0), jnp.float32(0.0))
        w_s[...] = jnp.dot(o2t, c_mat, preferred_element_type=jnp.float32)
        l0 = _rescal_losses(ent0[...], rel0f[...], h0[...], t0[...], nh0[...], nt0[...], r0[...])
        l1 = _rescal_losses(ent1[...], rel1f[...], h1[...], t1[...], nh1[...], nt1[...], r1[...])
        lane = jax.lax.broadcasted_iota(jnp.int32, (1, 128), 1)
        misc_s[...] = jnp.where(lane == 0, l0, jnp.where(lane == 1, l1, 0.0))

    rows_s[pl.ds(i % CHUNK, 1), :] = p_row[0]

    @pl.when(i % CHUNK == CHUNK - 1)
    def _chunk():
        chunk = i // CHUNK
        m = jnp.dot(rows_s[...], w_s[...], preferred_element_type=jnp.float32)
        u_chunk = u_s[pl.ds(chunk * CHUNK, CHUNK), :]
        acc_s[0, 0] += jnp.sum(m * u_chunk)

    @pl.when(i == NIDX - 1)
    def _final():
        lane = jax.lax.broadcasted_iota(jnp.int32, (1, 128), 1)
        w_loss = ALPHA * acc_s[0, 0]
        out[...] = jnp.where(lane == 2, w_loss, misc_s[...])


@jax.jit
def kernel(heads_0, tails_0, n_heads_0, n_tails_0, rels_0,
           heads_1, tails_1, n_heads_1, n_tails_1, rels_1,
           ent_emb_0, rel_emb_0, ent_emb_1, rel_emb_1, P):
    idx1 = jnp.concatenate([heads_0, tails_0, n_heads_0, n_tails_0]).astype(jnp.int32)
    idx2 = jnp.concatenate([heads_1, tails_1, n_heads_1, n_tails_1]).astype(jnp.int32)
    col = lambda x: jnp.reshape(x.astype(jnp.int32), (-1, 1))
    rel0f = jnp.reshape(rel_emb_0, (N_REL, DIM * DIM))
    rel1f = jnp.reshape(rel_emb_1, (N_REL, DIM * DIM))

    full = lambda shape: pl.BlockSpec(shape, lambda i, sp: (0,) * len(shape))
    grid_spec = pltpu.PrefetchScalarGridSpec(
        num_scalar_prefetch=1,
        grid=(NIDX,),
        in_specs=[
            pl.BlockSpec((1, 1, N_ENT), lambda i, sp: (sp[i], 0, 0)),
            full((N_ENT, DIM)), full((N_ENT, DIM)),
            full((N_REL, DIM * DIM)), full((N_REL, DIM * DIM)),
            full((NIDX, 1)), full((NIDX, 1)),
            full((B, 1)), full((B, 1)), full((B, 1)), full((B, 1)), full((B, 1)),
            full((B, 1)), full((B, 1)), full((B, 1)), full((B, 1)), full((B, 1)),
        ],
        out_specs=pl.BlockSpec((1, 128), lambda i, sp: (0, 0)),
        scratch_shapes=[
            pltpu.VMEM((N_ENT, 128), jnp.float32),
            pltpu.VMEM((NIDX, 128), jnp.float32),
            pltpu.VMEM((CHUNK, N_ENT), jnp.float32),
            pltpu.VMEM((1, 128), jnp.float32),
            pltpu.SMEM((1, 1), jnp.float32),
        ],
    )
    out = pl.pallas_call(
        _body,
        grid_spec=grid_spec,
        out_shape=jax.ShapeDtypeStruct((1, 128), jnp.float32),
    )(idx1, jnp.reshape(P, (N_ENT, 1, N_ENT)), ent_emb_0, ent_emb_1, rel0f, rel1f, col(idx1), col(idx2),
      col(heads_0), col(tails_0), col(n_heads_0), col(n_tails_0), col(rels_0),
      col(heads_1), col(tails_1), col(n_heads_1), col(n_tails_1), col(rels_1))
    return (out[0, :2], out[0, 2])


# speedup vs baseline: 5.8455x; 5.8455x over previous
import jax
import jax.numpy as jnp
from jax.experimental import pallas as pl
from jax.experimental.pallas import tpu as pltpu

N_ENT = 4096
N_REL = 200
DIM = 64
B = 128
NIDX = 4 * B
ALPHA = 0.1
MARGIN = 1.0
CHUNK = 64
NCHUNK = NIDX // CHUNK


def _onehot_rows(idx_col, n_cols):
    r = idx_col.shape[0]
    cols = jax.lax.broadcasted_iota(jnp.int32, (r, n_cols), 1)
    return jnp.where(cols == idx_col, jnp.float32(1.0), jnp.float32(0.0))


def _rescal_losses(ent, relf, h_i, t_i, nh_i, nt_i, r_i):
    h = jnp.dot(_onehot_rows(h_i, N_ENT), ent, preferred_element_type=jnp.float32)
    t = jnp.dot(_onehot_rows(t_i, N_ENT), ent, preferred_element_type=jnp.float32)
    nh = jnp.dot(_onehot_rows(nh_i, N_ENT), ent, preferred_element_type=jnp.float32)
    nt = jnp.dot(_onehot_rows(nt_i, N_ENT), ent, preferred_element_type=jnp.float32)
    rg = jnp.dot(_onehot_rows(r_i, N_REL), relf, preferred_element_type=jnp.float32)

    rows64 = jax.lax.broadcasted_iota(jnp.int32, (DIM, DIM * DIM), 0)
    colmod = jax.lax.broadcasted_iota(jnp.int32, (DIM, DIM * DIM), 1) % DIM
    tile_m = jnp.where(colmod == rows64, jnp.float32(1.0), jnp.float32(0.0))
    segrows = jax.lax.broadcasted_iota(jnp.int32, (DIM * DIM, DIM), 0) // DIM
    segcols = jax.lax.broadcasted_iota(jnp.int32, (DIM * DIM, DIM), 1)
    seg_m = jnp.where(segrows == segcols, jnp.float32(1.0), jnp.float32(0.0))

    def score(hv, tv):
        trep = jnp.dot(tv, tile_m, preferred_element_type=jnp.float32)
        tmp = jnp.dot(rg * trep, seg_m, preferred_element_type=jnp.float32)
        return jnp.sum(hv * tmp, axis=1)

    pos = score(h, t)
    neg = score(nh, nt)
    return jnp.mean(jax.nn.relu(MARGIN + neg - pos))


def _body(idx1_sm, p_hbm, ent0, ent1, rel0f, rel1f, idx1c, idx2c,
          h0, t0, nh0, nt0, r0, h1, t1, nh1, nt1, r1,
          out, rows_s, sems):

    def fire(c):
        buf = c % 2
        def fbody(k, _):
            row = idx1_sm[c * CHUNK + k]
            pltpu.make_async_copy(
                p_hbm.at[pl.ds(row, 1)], rows_s.at[buf, pl.ds(k, 1)],
                sems.at[buf]).start()
            return 0
        jax.lax.fori_loop(0, CHUNK, fbody, 0, unroll=8)

    def drain(c):
        buf = c % 2
        def wbody(k, _):
            pltpu.make_async_copy(
                p_hbm.at[pl.ds(0, 1)], rows_s.at[buf, pl.ds(k, 1)],
                sems.at[buf]).wait()
            return 0
        jax.lax.fori_loop(0, CHUNK, wbody, 0, unroll=8)

    fire(0)
    fire(1)

    o1 = _onehot_rows(idx1c[...], N_ENT)
    e1 = jnp.dot(o1, ent0[...], preferred_element_type=jnp.float32)
    o2 = _onehot_rows(idx2c[...], N_ENT)
    e2 = jnp.dot(o2, ent1[...], preferred_element_type=jnp.float32)
    a = jnp.sum(e1 * e1, axis=1, keepdims=True)
    b = jnp.sum(e2 * e2, axis=1, keepdims=True)
    ones = jnp.ones((NIDX, 1), jnp.float32)
    zeros = jnp.zeros((NIDX, 128 - 2 - DIM), jnp.float32)
    c_mat = jnp.concatenate([ones, b, e2, zeros], axis=1)
    u_mat = jnp.concatenate([a, ones, -2.0 * e1, zeros], axis=1)
    rows_iota = jax.lax.broadcasted_iota(jnp.int32, (N_ENT, NIDX), 0)
    o2t = jnp.where(rows_iota == jnp.reshape(idx2c[...], (1, NIDX)),
                    jnp.float32(1.0), jnp.float32(0.0))
    w_mat = jnp.dot(o2t, c_mat, preferred_element_type=jnp.float32)

    l0 = _rescal_losses(ent0[...], rel0f[...], h0[...], t0[...], nh0[...], nt0[...], r0[...])
    l1 = _rescal_losses(ent1[...], rel1f[...], h1[...], t1[...], nh1[...], nt1[...], r1[...])

    acc = jnp.float32(0.0)
    for c in range(NCHUNK):
        drain(c)
        if c + 2 < NCHUNK:
            fire(c + 2)
        m = jnp.dot(rows_s[c % 2], w_mat, preferred_element_type=jnp.float32)
        u_chunk = jax.lax.slice(u_mat, (c * CHUNK, 0), ((c + 1) * CHUNK, 128))
        acc = acc + jnp.sum(m * u_chunk)

    lane = jax.lax.broadcasted_iota(jnp.int32, (1, 128), 1)
    out[...] = jnp.where(lane == 0, l0,
                         jnp.where(lane == 1, l1,
                                   jnp.where(lane == 2, ALPHA * acc, 0.0)))


@jax.jit
def kernel(heads_0, tails_0, n_heads_0, n_tails_0, rels_0,
           heads_1, tails_1, n_heads_1, n_tails_1, rels_1,
           ent_emb_0, rel_emb_0, ent_emb_1, rel_emb_1, P):
    idx1 = jnp.concatenate([heads_0, tails_0, n_heads_0, n_tails_0]).astype(jnp.int32)
    idx2 = jnp.concatenate([heads_1, tails_1, n_heads_1, n_tails_1]).astype(jnp.int32)
    col = lambda x: jnp.reshape(x.astype(jnp.int32), (-1, 1))
    rel0f = jnp.reshape(rel_emb_0, (N_REL, DIM * DIM))
    rel1f = jnp.reshape(rel_emb_1, (N_REL, DIM * DIM))

    vmem = pl.BlockSpec(memory_space=pltpu.VMEM)
    out = pl.pallas_call(
        _body,
        in_specs=[
            pl.BlockSpec(memory_space=pltpu.SMEM),
            pl.BlockSpec(memory_space=pltpu.MemorySpace.HBM),
            vmem, vmem, vmem, vmem,
            vmem, vmem,
            vmem, vmem, vmem, vmem, vmem,
            vmem, vmem, vmem, vmem, vmem,
        ],
        out_specs=vmem,
        out_shape=jax.ShapeDtypeStruct((1, 128), jnp.float32),
        scratch_shapes=[
            pltpu.VMEM((2, CHUNK, N_ENT), jnp.float32),
            pltpu.SemaphoreType.DMA((2,)),
        ],
    )(idx1, P, ent_emb_0, ent_emb_1, rel0f, rel1f, col(idx1), col(idx2),
      col(heads_0), col(tails_0), col(n_heads_0), col(n_tails_0), col(rels_0),
      col(heads_1), col(tails_1), col(n_heads_1), col(n_tails_1), col(rels_1))
    return (out[0, :2], out[0, 2])
